# baseline (speedup 1.0000x reference)
import functools

import jax
import jax.numpy as jnp
from jax.experimental import pallas as pl
from jax.experimental.pallas import tpu as pltpu

DIM = 4096
NUM_CENTROIDS = 64
TOP_K = 2
TAU = 3.0
ENTROPY_WEIGHT = 0.001
DIVERSITY_WEIGHT = 0.001
N_TOKENS = 8192

BLOCK = 256


def _routing_kernel(x_ref, c_ref, gamma_ref,
                    y_ref, ent_ref, mp_ref, effk_ref, loss_ref,
                    cnorm_ref, cbf_ref, acc_ref):
    step = pl.program_id(0)
    nsteps = pl.num_programs(0)

    @pl.when(step == 0)
    def _init():
        c = c_ref[...]
        cn = c * jax.lax.rsqrt(
            jnp.maximum(jnp.sum(c * c, axis=1, keepdims=True), 1e-24))
        cnorm_ref[...] = cn.astype(jnp.bfloat16)
        cbf_ref[...] = c.astype(jnp.bfloat16)
        gram = jax.lax.dot_general(cn, cn, (((1,), (1,)), ((), ())),
                                   preferred_element_type=jnp.float32)
        ii = jax.lax.broadcasted_iota(jnp.int32, gram.shape, 0)
        jj = jax.lax.broadcasted_iota(jnp.int32, gram.shape, 1)
        off = jnp.where(ii == jj, 0.0, gram)
        acc_ref[0] = 0.0
        acc_ref[1] = 0.0
        acc_ref[2] = 0.0
        acc_ref[3] = jnp.sum(off * off) / (NUM_CENTROIDS * NUM_CENTROIDS)

    xb = x_ref[...]
    cn = cnorm_ref[...]

    inv = jax.lax.rsqrt(jnp.maximum(jnp.sum(xb * xb, axis=1, keepdims=True),
                                    1e-24))
    dots = jax.lax.dot_general(xb.astype(jnp.bfloat16), cn,
                               (((1,), (1,)), ((), ())),
                               preferred_element_type=jnp.float32)
    logits = (TAU * dots) * inv

    m = jnp.max(logits, axis=1, keepdims=True)
    e = jnp.exp(logits - m)
    s = jnp.sum(e, axis=1, keepdims=True)
    scores = e / s

    p = jnp.clip(scores, 1e-8, 1.0)
    ent_part = jnp.sum(-p * jnp.log(p))
    mp_part = jnp.sum(jnp.max(scores, axis=1))
    effk_part = jnp.sum(jnp.where(scores > 0.01, 1.0, 0.0))

    acc_ref[0] += ent_part
    acc_ref[1] += mp_part
    acc_ref[2] += effk_part

    col = jax.lax.broadcasted_iota(jnp.int32, scores.shape, 1)
    v1 = jnp.max(scores, axis=1, keepdims=True)
    j1 = jnp.min(jnp.where(scores == v1, col, NUM_CENTROIDS), axis=1,
                 keepdims=True)
    masked = jnp.where(col == j1, -jnp.inf, scores)
    v2 = jnp.max(masked, axis=1, keepdims=True)
    j2 = jnp.min(jnp.where(masked == v2, col, NUM_CENTROIDS), axis=1,
                 keepdims=True)

    scale = gamma_ref[0, 0] / (v1 + v2 + 1e-9)
    w = jnp.where(col == j1, v1 * scale, 0.0) + jnp.where(col == j2,
                                                          v2 * scale, 0.0)

    side = jax.lax.dot_general(w.astype(jnp.bfloat16), cbf_ref[...],
                               (((1,), (0,)), ((), ())),
                               preferred_element_type=jnp.float32)
    y_ref[...] = xb + side

    @pl.when(step == nsteps - 1)
    def _fini():
        ent = acc_ref[0] / N_TOKENS
        ent_ref[0, 0] = ent
        mp_ref[0, 0] = acc_ref[1] / N_TOKENS
        effk_ref[0, 0] = acc_ref[2] / N_TOKENS
        loss_ref[0, 0] = (ENTROPY_WEIGHT * (ent - 1.0) ** 2
                          + DIVERSITY_WEIGHT * acc_ref[3])


@functools.partial(jax.jit, static_argnames=())
def kernel(x, centroids, gamma):
    n, d = x.shape
    nsteps = n // BLOCK
    gamma_arr = jnp.asarray(gamma, jnp.float32).reshape(1, 1)
    scalar_spec = pl.BlockSpec((1, 1), lambda i: (0, 0),
                               memory_space=pltpu.SMEM)
    y, ent, mp, effk, loss = pl.pallas_call(
        _routing_kernel,
        grid=(nsteps,),
        in_specs=[
            pl.BlockSpec((BLOCK, d), lambda i: (i, 0)),
            pl.BlockSpec((NUM_CENTROIDS, d), lambda i: (0, 0)),
            scalar_spec,
        ],
        out_specs=[
            pl.BlockSpec((BLOCK, d), lambda i: (i, 0)),
            scalar_spec, scalar_spec, scalar_spec, scalar_spec,
        ],
        out_shape=[
            jax.ShapeDtypeStruct((n, d), jnp.float32),
            jax.ShapeDtypeStruct((1, 1), jnp.float32),
            jax.ShapeDtypeStruct((1, 1), jnp.float32),
            jax.ShapeDtypeStruct((1, 1), jnp.float32),
            jax.ShapeDtypeStruct((1, 1), jnp.float32),
        ],
        scratch_shapes=[
            pltpu.VMEM((NUM_CENTROIDS, d), jnp.bfloat16),
            pltpu.VMEM((NUM_CENTROIDS, d), jnp.bfloat16),
            pltpu.SMEM((4,), jnp.float32),
        ],
    )(x.astype(jnp.float32), centroids.astype(jnp.float32), gamma_arr)
    return (y, ent[0, 0], mp[0, 0], effk[0, 0], loss[0, 0])


# speedup vs baseline: 1.1257x; 1.1257x over previous
import functools

import jax
import jax.numpy as jnp
from jax.experimental import pallas as pl
from jax.experimental.pallas import tpu as pltpu

DIM = 4096
NUM_CENTROIDS = 64
TOP_K = 2
TAU = 3.0
ENTROPY_WEIGHT = 0.001
DIVERSITY_WEIGHT = 0.001
N_TOKENS = 8192

BLOCK = 512


def _routing_kernel(x_ref, c_ref, gamma_ref,
                    y_ref, ent_ref, mp_ref, effk_ref, loss_ref,
                    cnorm_ref, cbf_ref, acc_ref):
    step = pl.program_id(0)
    nsteps = pl.num_programs(0)

    @pl.when(step == 0)
    def _init():
        c = c_ref[...]
        cn = c * jax.lax.rsqrt(
            jnp.maximum(jnp.sum(c * c, axis=1, keepdims=True), 1e-24))
        cnorm_ref[...] = cn.astype(jnp.bfloat16)
        cbf_ref[...] = c.astype(jnp.bfloat16)
        gram = jax.lax.dot_general(cn, cn, (((1,), (1,)), ((), ())),
                                   preferred_element_type=jnp.float32)
        ii = jax.lax.broadcasted_iota(jnp.int32, gram.shape, 0)
        jj = jax.lax.broadcasted_iota(jnp.int32, gram.shape, 1)
        off = jnp.where(ii == jj, 0.0, gram)
        acc_ref[0] = 0.0
        acc_ref[1] = 0.0
        acc_ref[2] = 0.0
        acc_ref[3] = jnp.sum(off * off) / (NUM_CENTROIDS * NUM_CENTROIDS)

    xb = x_ref[...]
    cn = cnorm_ref[...]

    inv = jax.lax.rsqrt(jnp.maximum(jnp.sum(xb * xb, axis=1, keepdims=True),
                                    1e-24))
    dots = jax.lax.dot_general(xb.astype(jnp.bfloat16), cn,
                               (((1,), (1,)), ((), ())),
                               preferred_element_type=jnp.float32)
    logits = (TAU * dots) * inv

    m = jnp.max(logits, axis=1, keepdims=True)
    e = jnp.exp(logits - m)
    s = jnp.sum(e, axis=1, keepdims=True)
    scores = e / s

    p = jnp.clip(scores, 1e-8, 1.0)
    ent_part = jnp.sum(-p * jnp.log(p))
    mp_part = jnp.sum(jnp.max(scores, axis=1))
    effk_part = jnp.sum(jnp.where(scores > 0.01, 1.0, 0.0))

    acc_ref[0] += ent_part
    acc_ref[1] += mp_part
    acc_ref[2] += effk_part

    col = jax.lax.broadcasted_iota(jnp.int32, scores.shape, 1)
    v1 = jnp.max(scores, axis=1, keepdims=True)
    j1 = jnp.min(jnp.where(scores == v1, col, NUM_CENTROIDS), axis=1,
                 keepdims=True)
    masked = jnp.where(col == j1, -jnp.inf, scores)
    v2 = jnp.max(masked, axis=1, keepdims=True)
    j2 = jnp.min(jnp.where(masked == v2, col, NUM_CENTROIDS), axis=1,
                 keepdims=True)

    scale = gamma_ref[0, 0] / (v1 + v2 + 1e-9)
    w = jnp.where(col == j1, v1 * scale, 0.0) + jnp.where(col == j2,
                                                          v2 * scale, 0.0)

    side = jax.lax.dot_general(w.astype(jnp.bfloat16), cbf_ref[...],
                               (((1,), (0,)), ((), ())),
                               preferred_element_type=jnp.float32)
    y_ref[...] = xb + side

    @pl.when(step == nsteps - 1)
    def _fini():
        ent = acc_ref[0] / N_TOKENS
        ent_ref[0, 0] = ent
        mp_ref[0, 0] = acc_ref[1] / N_TOKENS
        effk_ref[0, 0] = acc_ref[2] / N_TOKENS
        loss_ref[0, 0] = (ENTROPY_WEIGHT * (ent - 1.0) ** 2
                          + DIVERSITY_WEIGHT * acc_ref[3])


@functools.partial(jax.jit, static_argnames=())
def kernel(x, centroids, gamma):
    n, d = x.shape
    nsteps = n // BLOCK
    gamma_arr = jnp.asarray(gamma, jnp.float32).reshape(1, 1)
    scalar_spec = pl.BlockSpec((1, 1), lambda i: (0, 0),
                               memory_space=pltpu.SMEM)
    y, ent, mp, effk, loss = pl.pallas_call(
        _routing_kernel,
        grid=(nsteps,),
        in_specs=[
            pl.BlockSpec((BLOCK, d), lambda i: (i, 0)),
            pl.BlockSpec((NUM_CENTROIDS, d), lambda i: (0, 0)),
            scalar_spec,
        ],
        out_specs=[
            pl.BlockSpec((BLOCK, d), lambda i: (i, 0)),
            scalar_spec, scalar_spec, scalar_spec, scalar_spec,
        ],
        out_shape=[
            jax.ShapeDtypeStruct((n, d), jnp.float32),
            jax.ShapeDtypeStruct((1, 1), jnp.float32),
            jax.ShapeDtypeStruct((1, 1), jnp.float32),
            jax.ShapeDtypeStruct((1, 1), jnp.float32),
            jax.ShapeDtypeStruct((1, 1), jnp.float32),
        ],
        scratch_shapes=[
            pltpu.VMEM((NUM_CENTROIDS, d), jnp.bfloat16),
            pltpu.VMEM((NUM_CENTROIDS, d), jnp.bfloat16),
            pltpu.SMEM((4,), jnp.float32),
        ],
    )(x.astype(jnp.float32), centroids.astype(jnp.float32), gamma_arr)
    return (y, ent[0, 0], mp[0, 0], effk[0, 0], loss[0, 0])


# speedup vs baseline: 1.1750x; 1.0437x over previous
import functools

import jax
import jax.numpy as jnp
from jax.experimental import pallas as pl
from jax.experimental.pallas import tpu as pltpu

DIM = 4096
NUM_CENTROIDS = 64
TAU = 3.0
ENTROPY_WEIGHT = 0.001
DIVERSITY_WEIGHT = 0.001
N_TOKENS = 8192

BLOCK = 512
NB = N_TOKENS // BLOCK
DEPTH = 3


def _compute_block(xb, cn, cbf, gamma, acc_ref):
    inv = jax.lax.rsqrt(jnp.maximum(jnp.sum(xb * xb, axis=1, keepdims=True),
                                    1e-24))
    dots = jax.lax.dot_general(xb.astype(jnp.bfloat16), cn,
                               (((1,), (1,)), ((), ())),
                               preferred_element_type=jnp.float32)
    logits = (TAU * dots) * inv
    m = jnp.max(logits, axis=1, keepdims=True)
    e = jnp.exp(logits - m)
    scores = e / jnp.sum(e, axis=1, keepdims=True)

    p = jnp.clip(scores, 1e-8, 1.0)
    acc_ref[0] += jnp.sum(-p * jnp.log(p))
    acc_ref[1] += jnp.sum(jnp.max(scores, axis=1))
    acc_ref[2] += jnp.sum(jnp.where(scores > 0.01, 1.0, 0.0))

    col = jax.lax.broadcasted_iota(jnp.int32, scores.shape, 1)
    v1 = jnp.max(scores, axis=1, keepdims=True)
    j1 = jnp.min(jnp.where(scores == v1, col, NUM_CENTROIDS), axis=1,
                 keepdims=True)
    masked = jnp.where(col == j1, -jnp.inf, scores)
    v2 = jnp.max(masked, axis=1, keepdims=True)
    j2 = jnp.min(jnp.where(masked == v2, col, NUM_CENTROIDS), axis=1,
                 keepdims=True)
    scale = gamma / (v1 + v2 + 1e-9)
    w = jnp.where(col == j1, v1 * scale, 0.0) + jnp.where(col == j2,
                                                          v2 * scale, 0.0)
    side = jax.lax.dot_general(w.astype(jnp.bfloat16), cbf,
                               (((1,), (0,)), ((), ())),
                               preferred_element_type=jnp.float32)
    return xb + side


def _routing_kernel(x_hbm, c_ref, gamma_ref,
                    y_hbm, ent_ref, mp_ref, effk_ref, loss_ref,
                    xbuf, ybuf, cnorm_ref, cbf_ref, acc_ref,
                    in_sem, out_sem):
    c = c_ref[...]
    cn32 = c * jax.lax.rsqrt(
        jnp.maximum(jnp.sum(c * c, axis=1, keepdims=True), 1e-24))
    cnorm_ref[...] = cn32.astype(jnp.bfloat16)
    cbf_ref[...] = c.astype(jnp.bfloat16)
    gram = jax.lax.dot_general(cn32, cn32, (((1,), (1,)), ((), ())),
                               preferred_element_type=jnp.float32)
    ii = jax.lax.broadcasted_iota(jnp.int32, gram.shape, 0)
    jj = jax.lax.broadcasted_iota(jnp.int32, gram.shape, 1)
    off = jnp.where(ii == jj, 0.0, gram)
    acc_ref[0] = 0.0
    acc_ref[1] = 0.0
    acc_ref[2] = 0.0
    acc_ref[3] = jnp.sum(off * off) / (NUM_CENTROIDS * NUM_CENTROIDS)

    def copy_in(i, slot):
        return pltpu.make_async_copy(
            x_hbm.at[pl.ds(i * BLOCK, BLOCK)], xbuf.at[slot], in_sem)

    def copy_out(i, slot):
        return pltpu.make_async_copy(
            ybuf.at[slot], y_hbm.at[pl.ds(i * BLOCK, BLOCK)], out_sem)

    for s in range(DEPTH):
        copy_in(s, s).start()

    gamma = gamma_ref[0, 0]
    cn = cnorm_ref[...]
    cbf = cbf_ref[...]

    for i in range(NB):
        slot = i % DEPTH
        copy_in(i, slot).wait()
        if i >= DEPTH:
            copy_out(i - DEPTH, slot).wait()
        ybuf[slot] = _compute_block(xbuf[slot], cn, cbf, gamma, acc_ref)
        copy_out(i, slot).start()
        if i + DEPTH < NB:
            copy_in(i + DEPTH, slot).start()

    for i in range(NB - DEPTH, NB):
        copy_out(i, i % DEPTH).wait()

    ent = acc_ref[0] / N_TOKENS
    ent_ref[0, 0] = ent
    mp_ref[0, 0] = acc_ref[1] / N_TOKENS
    effk_ref[0, 0] = acc_ref[2] / N_TOKENS
    loss_ref[0, 0] = (ENTROPY_WEIGHT * (ent - 1.0) ** 2
                      + DIVERSITY_WEIGHT * acc_ref[3])


@functools.partial(jax.jit, static_argnames=())
def kernel(x, centroids, gamma):
    n, d = x.shape
    gamma_arr = jnp.asarray(gamma, jnp.float32).reshape(1, 1)
    scalar_spec = pl.BlockSpec(memory_space=pltpu.SMEM)
    y, ent, mp, effk, loss = pl.pallas_call(
        _routing_kernel,
        in_specs=[
            pl.BlockSpec(memory_space=pltpu.HBM),
            pl.BlockSpec(memory_space=pltpu.VMEM),
            scalar_spec,
        ],
        out_specs=[
            pl.BlockSpec(memory_space=pltpu.HBM),
            scalar_spec, scalar_spec, scalar_spec, scalar_spec,
        ],
        out_shape=[
            jax.ShapeDtypeStruct((n, d), jnp.float32),
            jax.ShapeDtypeStruct((1, 1), jnp.float32),
            jax.ShapeDtypeStruct((1, 1), jnp.float32),
            jax.ShapeDtypeStruct((1, 1), jnp.float32),
            jax.ShapeDtypeStruct((1, 1), jnp.float32),
        ],
        scratch_shapes=[
            pltpu.VMEM((DEPTH, BLOCK, DIM), jnp.float32),
            pltpu.VMEM((DEPTH, BLOCK, DIM), jnp.float32),
            pltpu.VMEM((NUM_CENTROIDS, DIM), jnp.bfloat16),
            pltpu.VMEM((NUM_CENTROIDS, DIM), jnp.bfloat16),
            pltpu.SMEM((4,), jnp.float32),
            pltpu.SemaphoreType.DMA,
            pltpu.SemaphoreType.DMA,
        ],
    )(x.astype(jnp.float32), centroids.astype(jnp.float32), gamma_arr)
    return (y, ent[0, 0], mp[0, 0], effk[0, 0], loss[0, 0])


# speedup vs baseline: 1.1901x; 1.0129x over previous
import functools

import jax
import jax.numpy as jnp
from jax.experimental import pallas as pl
from jax.experimental.pallas import tpu as pltpu

DIM = 4096
NUM_CENTROIDS = 64
TAU = 3.0
ENTROPY_WEIGHT = 0.001
DIVERSITY_WEIGHT = 0.001
N_TOKENS = 8192

BLOCK = 512
SCHED = [256] + [512] * 15 + [256]
OFFS = [sum(SCHED[:i]) for i in range(len(SCHED))]
NB = len(SCHED)
IN_DEPTH = 4
OUT_DEPTH = 2


def _compute_block(xb, cn, cbf, gamma, acc_ref):
    inv = jax.lax.rsqrt(jnp.maximum(jnp.sum(xb * xb, axis=1, keepdims=True),
                                    1e-24))
    dots = jax.lax.dot_general(xb.astype(jnp.bfloat16), cn,
                               (((1,), (1,)), ((), ())),
                               preferred_element_type=jnp.float32)
    logits = (TAU * dots) * inv
    m = jnp.max(logits, axis=1, keepdims=True)
    e = jnp.exp(logits - m)
    scores = e / jnp.sum(e, axis=1, keepdims=True)

    p = jnp.clip(scores, 1e-8, 1.0)
    acc_ref[0] += jnp.sum(-p * jnp.log(p))
    acc_ref[1] += jnp.sum(jnp.max(scores, axis=1))
    acc_ref[2] += jnp.sum(jnp.where(scores > 0.01, 1.0, 0.0))

    col = jax.lax.broadcasted_iota(jnp.int32, scores.shape, 1)
    v1 = jnp.max(scores, axis=1, keepdims=True)
    j1 = jnp.min(jnp.where(scores == v1, col, NUM_CENTROIDS), axis=1,
                 keepdims=True)
    masked = jnp.where(col == j1, -jnp.inf, scores)
    v2 = jnp.max(masked, axis=1, keepdims=True)
    j2 = jnp.min(jnp.where(masked == v2, col, NUM_CENTROIDS), axis=1,
                 keepdims=True)
    scale = gamma / (v1 + v2 + 1e-9)
    w = jnp.where(col == j1, v1 * scale, 0.0) + jnp.where(col == j2,
                                                          v2 * scale, 0.0)
    side = jax.lax.dot_general(w.astype(jnp.bfloat16), cbf,
                               (((1,), (0,)), ((), ())),
                               preferred_element_type=jnp.float32)
    return xb + side


def _routing_kernel(x_hbm, c_ref, gamma_ref,
                    y_hbm, ent_ref, mp_ref, effk_ref, loss_ref,
                    xbuf, ybuf, cnorm_ref, cbf_ref, acc_ref,
                    in_sem, out_sem):
    c = c_ref[...]
    cn32 = c * jax.lax.rsqrt(
        jnp.maximum(jnp.sum(c * c, axis=1, keepdims=True), 1e-24))
    cnorm_ref[...] = cn32.astype(jnp.bfloat16)
    cbf_ref[...] = c.astype(jnp.bfloat16)
    gram = jax.lax.dot_general(cn32, cn32, (((1,), (1,)), ((), ())),
                               preferred_element_type=jnp.float32)
    ii = jax.lax.broadcasted_iota(jnp.int32, gram.shape, 0)
    jj = jax.lax.broadcasted_iota(jnp.int32, gram.shape, 1)
    off = jnp.where(ii == jj, 0.0, gram)
    acc_ref[0] = 0.0
    acc_ref[1] = 0.0
    acc_ref[2] = 0.0
    acc_ref[3] = jnp.sum(off * off) / (NUM_CENTROIDS * NUM_CENTROIDS)

    def copy_in(i):
        nr = SCHED[i]
        return pltpu.make_async_copy(
            x_hbm.at[pl.ds(OFFS[i], nr)],
            xbuf.at[i % IN_DEPTH, pl.ds(0, nr)], in_sem)

    def copy_out(i):
        nr = SCHED[i]
        return pltpu.make_async_copy(
            ybuf.at[i % OUT_DEPTH, pl.ds(0, nr)],
            y_hbm.at[pl.ds(OFFS[i], nr)], out_sem)

    for s in range(IN_DEPTH):
        copy_in(s).start()

    gamma = gamma_ref[0, 0]
    cn = cnorm_ref[...]
    cbf = cbf_ref[...]

    for i in range(NB):
        copy_in(i).wait()
        if i >= OUT_DEPTH:
            copy_out(i - OUT_DEPTH).wait()
        nr = SCHED[i]
        ybuf[i % OUT_DEPTH, pl.ds(0, nr)] = _compute_block(
            xbuf[i % IN_DEPTH, pl.ds(0, nr)], cn, cbf, gamma, acc_ref)
        copy_out(i).start()
        if i + IN_DEPTH < NB:
            copy_in(i + IN_DEPTH).start()

    for i in range(NB - OUT_DEPTH, NB):
        copy_out(i).wait()

    ent = acc_ref[0] / N_TOKENS
    ent_ref[0, 0] = ent
    mp_ref[0, 0] = acc_ref[1] / N_TOKENS
    effk_ref[0, 0] = acc_ref[2] / N_TOKENS
    loss_ref[0, 0] = (ENTROPY_WEIGHT * (ent - 1.0) ** 2
                      + DIVERSITY_WEIGHT * acc_ref[3])


@functools.partial(jax.jit, static_argnames=())
def kernel(x, centroids, gamma):
    n, d = x.shape
    gamma_arr = jnp.asarray(gamma, jnp.float32).reshape(1, 1)
    scalar_spec = pl.BlockSpec(memory_space=pltpu.SMEM)
    y, ent, mp, effk, loss = pl.pallas_call(
        _routing_kernel,
        in_specs=[
            pl.BlockSpec(memory_space=pltpu.HBM),
            pl.BlockSpec(memory_space=pltpu.VMEM),
            scalar_spec,
        ],
        out_specs=[
            pl.BlockSpec(memory_space=pltpu.HBM),
            scalar_spec, scalar_spec, scalar_spec, scalar_spec,
        ],
        out_shape=[
            jax.ShapeDtypeStruct((n, d), jnp.float32),
            jax.ShapeDtypeStruct((1, 1), jnp.float32),
            jax.ShapeDtypeStruct((1, 1), jnp.float32),
            jax.ShapeDtypeStruct((1, 1), jnp.float32),
            jax.ShapeDtypeStruct((1, 1), jnp.float32),
        ],
        scratch_shapes=[
            pltpu.VMEM((IN_DEPTH, BLOCK, DIM), jnp.float32),
            pltpu.VMEM((OUT_DEPTH, BLOCK, DIM), jnp.float32),
            pltpu.VMEM((NUM_CENTROIDS, DIM), jnp.bfloat16),
            pltpu.VMEM((NUM_CENTROIDS, DIM), jnp.bfloat16),
            pltpu.SMEM((4,), jnp.float32),
            pltpu.SemaphoreType.DMA,
            pltpu.SemaphoreType.DMA,
        ],
    )(x.astype(jnp.float32), centroids.astype(jnp.float32), gamma_arr)
    return (y, ent[0, 0], mp[0, 0], effk[0, 0], loss[0, 0])
